# baseline (speedup 1.0000x reference)
import functools

import jax
import jax.numpy as jnp
from jax import lax
from jax.experimental import pallas as pl
from jax.experimental.pallas import tpu as pltpu
from jax.experimental.pallas import tpu_sc as plsc

N = 10000
E = 320000
D = 128
NC = 2
NS = 16
NW = NC * NS
EPW = E // NW
C = 80
G = 16
NGROUP = C // G
NCHUNK = EPW // C
ROWS_PT = 624
ROWS_TAIL = N - ROWS_PT * NS

_MESH = plsc.VectorSubcoreMesh(core_axis_name="c", subcore_axis_name="s")


def _edge_body(xl_hbm, xr_hbm, src_hbm, dst_hbm, att_hbm, zeros_hbm, zn_hbm,
               out_hbm, den_hbm,
               acc_sh, den_sh,
               xlr2, xrr2, srcx2, dstx2, av2, sdst2,
               attv, pbuf, abuf,
               g1, g2, s1, s2, ix):
    c = lax.axis_index("c")
    s = lax.axis_index("s")
    wid = c * NS + s

    pltpu.sync_copy(zeros_hbm.at[pl.ds(s * ROWS_PT, ROWS_PT)],
                    acc_sh.at[pl.ds(s * ROWS_PT, ROWS_PT)])
    pltpu.sync_copy(zn_hbm.at[pl.ds(s * ROWS_PT, ROWS_PT)],
                    den_sh.at[pl.ds(s * ROWS_PT, ROWS_PT)])

    @pl.when(s == NS - 1)
    def _():
        pltpu.sync_copy(zeros_hbm.at[pl.ds(NS * ROWS_PT, ROWS_TAIL)],
                        acc_sh.at[pl.ds(NS * ROWS_PT, ROWS_TAIL)])
        pltpu.sync_copy(zn_hbm.at[pl.ds(NS * ROWS_PT, ROWS_TAIL)],
                        den_sh.at[pl.ds(NS * ROWS_PT, ROWS_TAIL)])

    pltpu.sync_copy(att_hbm, attv)
    plsc.subcore_barrier()

    att_regs = [attv[pl.ds(16 * k, 16)] for k in range(8)]
    iota16 = lax.iota(jnp.int32, 16)
    splats = [jnp.full((16,), k, jnp.int32) for k in range(G)]

    base = wid * EPW

    def issue_idx(i, b):
        pltpu.async_copy(src_hbm.at[pl.ds(base + i * C, C)], srcx2.at[b],
                         ix.at[b])
        pltpu.async_copy(dst_hbm.at[pl.ds(base + i * C, C)], dstx2.at[b],
                         ix.at[b])

    def wait_idx(b):
        pltpu.make_async_copy(src_hbm.at[pl.ds(0, C)], srcx2.at[b],
                              ix.at[b]).wait()
        pltpu.make_async_copy(dst_hbm.at[pl.ds(0, C)], dstx2.at[b],
                              ix.at[b]).wait()

    def issue_gathers(b):
        pltpu.async_copy(xl_hbm.at[srcx2.at[b]], xlr2.at[b], g1.at[b])
        pltpu.async_copy(xr_hbm.at[dstx2.at[b]], xrr2.at[b], g2.at[b])

    def wait_gathers(b):
        pltpu.make_async_copy(xl_hbm.at[pl.ds(0, C)], xlr2.at[b],
                              g1.at[b]).wait()
        pltpu.make_async_copy(xr_hbm.at[pl.ds(0, C)], xrr2.at[b],
                              g2.at[b]).wait()

    def wait_scatter(b):
        pltpu.make_async_copy(xl_hbm.at[pl.ds(0, C)], xlr2.at[b],
                              s1.at[b]).wait()
        pltpu.make_async_copy(zn_hbm.at[pl.ds(0, C)], av2.at[b],
                              s2.at[b]).wait()

    def issue_scatter(b):
        pltpu.async_copy(xlr2.at[b], acc_sh.at[sdst2.at[b]], s1.at[b],
                         add=True)
        pltpu.async_copy(av2.at[b], den_sh.at[sdst2.at[b]], s2.at[b],
                         add=True)

    def compute_chunk(b):
        for t in range(C // 16):
            sdst2[b, pl.ds(16 * t, 16)] = dstx2[b, pl.ds(16 * t, 16)]

        def group_body(j, carry2):
            for k in range(G):
                row = j * G + k
                acc = None
                for q in range(8):
                    sq = (xlr2[b, row, pl.ds(16 * q, 16)] +
                          xrr2[b, row, pl.ds(16 * q, 16)])
                    eq = jnp.maximum(sq, 0.2 * sq)
                    t = eq * att_regs[q]
                    acc = t if acc is None else acc + t
                pbuf[k, :] = acc
            cols = [plsc.load_gather(pbuf, [iota16, splats[jc]])
                    for jc in range(16)]
            while len(cols) > 1:
                cols = [cols[i] + cols[i + 1] for i in range(0, len(cols), 2)]
            aexp = jnp.exp(cols[0])
            av2[b, pl.ds(j * G, 16)] = aexp
            for jc in range(16):
                plsc.store_scatter(abuf, [iota16, splats[jc]], aexp)
            for k in range(G):
                row = j * G + k
                a16 = abuf[k, :]
                for q in range(8):
                    xlr2[b, row, pl.ds(16 * q, 16)] = (
                        xlr2[b, row, pl.ds(16 * q, 16)] * a16)
            return carry2

        lax.fori_loop(0, NGROUP, group_body, 0)

    issue_idx(0, 0)
    wait_idx(0)
    issue_idx(1, 1)
    issue_gathers(0)

    def chunk_body(i, carry):
        b = lax.rem(i, 2)
        nb = 1 - b

        @pl.when(i + 1 < NCHUNK)
        def _():
            wait_idx(nb)

        @pl.when(i > 0)
        def _():
            wait_scatter(nb)

        @pl.when(i + 1 < NCHUNK)
        def _():
            issue_gathers(nb)

        wait_gathers(b)

        @pl.when(b == 0)
        def _():
            compute_chunk(0)

        @pl.when(b == 1)
        def _():
            compute_chunk(1)

        issue_scatter(b)

        @pl.when(i + 2 < NCHUNK)
        def _():
            issue_idx(i + 2, b)
        return carry

    lax.fori_loop(0, NCHUNK, chunk_body, 0)
    wait_scatter(lax.rem(NCHUNK - 1, 2))

    plsc.subcore_barrier()
    pltpu.sync_copy(acc_sh.at[pl.ds(s * ROWS_PT, ROWS_PT)],
                    out_hbm.at[c, pl.ds(s * ROWS_PT, ROWS_PT)])
    pltpu.sync_copy(den_sh.at[pl.ds(s * ROWS_PT, ROWS_PT)],
                    den_hbm.at[c, pl.ds(s * ROWS_PT, ROWS_PT)])

    @pl.when(s == NS - 1)
    def _():
        pltpu.sync_copy(acc_sh.at[pl.ds(NS * ROWS_PT, ROWS_TAIL)],
                        out_hbm.at[c, pl.ds(NS * ROWS_PT, ROWS_TAIL)])
        pltpu.sync_copy(den_sh.at[pl.ds(NS * ROWS_PT, ROWS_TAIL)],
                        den_hbm.at[c, pl.ds(NS * ROWS_PT, ROWS_TAIL)])


_edge_pass = functools.partial(
    pl.kernel,
    out_type=(jax.ShapeDtypeStruct((NC, N, D), jnp.float32),
              jax.ShapeDtypeStruct((NC, N), jnp.float32)),
    mesh=_MESH,
    scratch_types=[
        pltpu.VMEM_SHARED((N, D), jnp.float32),
        pltpu.VMEM_SHARED((N,), jnp.float32),
        pltpu.VMEM((2, C, D), jnp.float32),
        pltpu.VMEM((2, C, D), jnp.float32),
        pltpu.VMEM((2, C), jnp.int32),
        pltpu.VMEM((2, C), jnp.int32),
        pltpu.VMEM((2, C), jnp.float32),
        pltpu.VMEM((2, C), jnp.int32),
        pltpu.VMEM((D,), jnp.float32),
        pltpu.VMEM((G, 16), jnp.float32),
        pltpu.VMEM((16, 16), jnp.float32),
        pltpu.SemaphoreType.DMA((2,)),
        pltpu.SemaphoreType.DMA((2,)),
        pltpu.SemaphoreType.DMA((2,)),
        pltpu.SemaphoreType.DMA((2,)),
        pltpu.SemaphoreType.DMA((2,)),
    ],
    compiler_params=pltpu.CompilerParams(needs_layout_passes=False,
                                         use_tc_tiling_on_sc=False),
)(_edge_body)


def _mm(a, b):
    return jax.lax.dot(a, b)


_BR = 1000
_NBLK = N // _BR


def _full(shape):
    return pl.BlockSpec(shape, lambda i: tuple(0 for _ in shape))


def _rows(width):
    return pl.BlockSpec((_BR, width), lambda i: (i, 0))


def _tc_pre_body(x_ref, Win_ref, bin_ref, Wl_ref, bl_ref, Wr_ref, br_ref,
                 xl_ref, xr_ref):
    h = jnp.maximum(_mm(x_ref[...], Win_ref[...]) + bin_ref[...], 0.0)
    xl_ref[...] = _mm(h, Wl_ref[...]) + bl_ref[...]
    xr_ref[...] = _mm(h, Wr_ref[...]) + br_ref[...]


def _tc_pre(x, W_in, b_in, Wl, bl, Wr, br):
    return pl.pallas_call(
        _tc_pre_body,
        grid=(_NBLK,),
        in_specs=[_rows(D), _full((D, D)), _full((1, D)), _full((D, D)),
                  _full((1, D)), _full((D, D)), _full((1, D))],
        out_specs=[_rows(D), _rows(D)],
        out_shape=[jax.ShapeDtypeStruct((N, D), jnp.float32)] * 2,
    )(x, W_in, b_in.reshape(1, D), Wl, bl.reshape(1, D), Wr, br.reshape(1, D))


def _normalize(acc_ref, den_ref, bias_ref, g_ref, bt_ref, res, do_relu):
    num = acc_ref[0] + acc_ref[1]
    den = den_ref[0] + den_ref[1]
    h = num / (den + 1e-16) + bias_ref[...]
    if res is not None:
        h = h + res
    mu = jnp.mean(h, axis=-1, keepdims=True)
    var = jnp.mean((h - mu) ** 2, axis=-1, keepdims=True)
    h = (h - mu) / jnp.sqrt(var + 1e-5) * g_ref[...] + bt_ref[...]
    if do_relu:
        h = jnp.maximum(h, 0.0)
    return h


def _tc_mid_body(has_res, args):
    if has_res:
        (acc_ref, den_ref, bias_ref, g_ref, bt_ref, res_ref, Wl_ref, bl_ref,
         Wr_ref, br_ref, h_ref, xl_ref, xr_ref) = args
        res = res_ref[...]
    else:
        (acc_ref, den_ref, bias_ref, g_ref, bt_ref, Wl_ref, bl_ref, Wr_ref,
         br_ref, h_ref, xl_ref, xr_ref) = args
        res = None
    h = _normalize(acc_ref, den_ref, bias_ref, g_ref, bt_ref, res, True)
    h_ref[...] = h
    xl_ref[...] = _mm(h, Wl_ref[...]) + bl_ref[...]
    xr_ref[...] = _mm(h, Wr_ref[...]) + br_ref[...]


_ACC_SPEC = pl.BlockSpec((NC, _BR, D), lambda i: (0, i, 0))
_DEN_SPEC = pl.BlockSpec((NC, _BR, 1), lambda i: (0, i, 0))


def _tc_mid(acc, den, bias, g, bt, res, Wl, bl, Wr, br):
    has_res = res is not None
    in_specs = [_ACC_SPEC, _DEN_SPEC,
                _full((1, D)), _full((1, D)), _full((1, D))]
    ops = [acc, den.reshape(NC, N, 1), bias.reshape(1, D), g.reshape(1, D),
           bt.reshape(1, D)]
    if has_res:
        in_specs.append(_rows(D))
        ops.append(res)
    in_specs += [_full((D, D)), _full((1, D)), _full((D, D)), _full((1, D))]
    ops += [Wl, bl.reshape(1, D), Wr, br.reshape(1, D)]
    return pl.pallas_call(
        lambda *args: _tc_mid_body(has_res, args),
        grid=(_NBLK,),
        in_specs=in_specs,
        out_specs=[_rows(D)] * 3,
        out_shape=[jax.ShapeDtypeStruct((N, D), jnp.float32)] * 3,
    )(*ops)


def _tc_post_body(acc_ref, den_ref, bias_ref, g_ref, bt_ref, res_ref,
                  Wo1_ref, bo1_ref, Wo2_ref, bo2_ref, out_ref):
    h = _normalize(acc_ref, den_ref, bias_ref, g_ref, bt_ref, res_ref[...],
                   False)
    t = jnp.maximum(_mm(h, Wo1_ref[...]) + bo1_ref[...], 0.0)
    out_ref[...] = _mm(t, Wo2_ref[...]) + bo2_ref[...]


def _tc_post(acc, den, bias, g, bt, res, Wo1, bo1, Wo2, bo2):
    return pl.pallas_call(
        _tc_post_body,
        grid=(_NBLK,),
        in_specs=[_ACC_SPEC, _DEN_SPEC,
                  _full((1, D)), _full((1, D)), _full((1, D)), _rows(D),
                  _full((D, D // 2)), _full((1, D // 2)),
                  _full((D // 2, 2)), _full((1, 2))],
        out_specs=_rows(2),
        out_shape=jax.ShapeDtypeStruct((N, 2), jnp.float32),
    )(acc, den.reshape(NC, N, 1), bias.reshape(1, D), g.reshape(1, D),
      bt.reshape(1, D), res, Wo1, bo1.reshape(1, D // 2), Wo2,
      bo2.reshape(1, 2))


def kernel(x, edge_index, W_in, b_in, Wl0, bl0, Wr0, br0, att0, bias0, g0, bt0,
           Wl1, bl1, Wr1, br1, att1, bias1, g1, bt1,
           Wl2, bl2, Wr2, br2, att2, bias2, g2, bt2, Wo1, bo1, Wo2, bo2):
    src = edge_index[0]
    dst = edge_index[1]
    zeros = jnp.zeros((N, D), jnp.float32)
    zn = jnp.zeros((N,), jnp.float32)

    xl, xr = _tc_pre(x, W_in, b_in, Wl0, bl0, Wr0, br0)
    acc0, den0 = _edge_pass(xl, xr, src, dst, att0[0], zeros, zn)
    h1, xl1, xr1 = _tc_mid(acc0, den0, bias0, g0, bt0, None, Wl1, bl1, Wr1, br1)
    acc1, den1 = _edge_pass(xl1, xr1, src, dst, att1[0], zeros, zn)
    h2, xl2, xr2 = _tc_mid(acc1, den1, bias1, g1, bt1, h1, Wl2, bl2, Wr2, br2)
    acc2, den2 = _edge_pass(xl2, xr2, src, dst, att2[0], zeros, zn)
    return _tc_post(acc2, den2, bias2, g2, bt2, h2, Wo1, bo1, Wo2, bo2)


# speedup vs baseline: 2.0753x; 2.0753x over previous
import functools

import jax
import jax.numpy as jnp
from jax import lax
from jax.experimental import pallas as pl
from jax.experimental.pallas import tpu as pltpu
from jax.experimental.pallas import tpu_sc as plsc

N = 10000
E = 320000
D = 128
NC = 2
NS = 16
NW = NC * NS
EPW = E // NW
C = 80
G = 16
NGROUP = C // G
NCHUNK = EPW // C
ROWS_PT = 624
ROWS_TAIL = N - ROWS_PT * NS

_MESH = plsc.VectorSubcoreMesh(core_axis_name="c", subcore_axis_name="s")


def _edge_body(xl_hbm, xr_hbm, src_hbm, dst_hbm, att_hbm, zeros_hbm, zn_hbm,
               out_hbm, den_hbm,
               acc_sh, den_sh,
               xlr2, xrr2, srcx2, dstx2, av2, sdst2,
               attv, pbuf, abuf,
               g1, g2, s1, s2, ix):
    c = lax.axis_index("c")
    s = lax.axis_index("s")
    wid = c * NS + s

    pltpu.sync_copy(zeros_hbm.at[pl.ds(s * ROWS_PT, ROWS_PT)],
                    acc_sh.at[pl.ds(s * ROWS_PT, ROWS_PT)])
    pltpu.sync_copy(zn_hbm.at[pl.ds(s * ROWS_PT, ROWS_PT)],
                    den_sh.at[pl.ds(s * ROWS_PT, ROWS_PT)])

    @pl.when(s == NS - 1)
    def _():
        pltpu.sync_copy(zeros_hbm.at[pl.ds(NS * ROWS_PT, ROWS_TAIL)],
                        acc_sh.at[pl.ds(NS * ROWS_PT, ROWS_TAIL)])
        pltpu.sync_copy(zn_hbm.at[pl.ds(NS * ROWS_PT, ROWS_TAIL)],
                        den_sh.at[pl.ds(NS * ROWS_PT, ROWS_TAIL)])

    pltpu.sync_copy(att_hbm, attv)
    plsc.subcore_barrier()

    att_regs = [attv[pl.ds(16 * k, 16)] for k in range(8)]
    iota16 = lax.iota(jnp.int32, 16)
    splats = [jnp.full((16,), k, jnp.int32) for k in range(G)]

    base = wid * EPW

    def issue_idx(i, b):
        pltpu.async_copy(src_hbm.at[pl.ds(base + i * C, C)], srcx2.at[b],
                         ix.at[b])
        pltpu.async_copy(dst_hbm.at[pl.ds(base + i * C, C)], dstx2.at[b],
                         ix.at[b])

    def wait_idx(b):
        pltpu.make_async_copy(src_hbm.at[pl.ds(0, C)], srcx2.at[b],
                              ix.at[b]).wait()
        pltpu.make_async_copy(dst_hbm.at[pl.ds(0, C)], dstx2.at[b],
                              ix.at[b]).wait()

    def issue_gathers(b):
        pltpu.async_copy(xl_hbm.at[srcx2.at[b]], xlr2.at[b], g1.at[b])
        pltpu.async_copy(xr_hbm.at[dstx2.at[b]], xrr2.at[b], g2.at[b])

    def wait_gathers(b):
        pltpu.make_async_copy(xl_hbm.at[pl.ds(0, C)], xlr2.at[b],
                              g1.at[b]).wait()
        pltpu.make_async_copy(xr_hbm.at[pl.ds(0, C)], xrr2.at[b],
                              g2.at[b]).wait()

    def wait_scatter(b):
        pltpu.make_async_copy(xl_hbm.at[pl.ds(0, C)], xlr2.at[b],
                              s1.at[b]).wait()
        pltpu.make_async_copy(zn_hbm.at[pl.ds(0, C)], av2.at[b],
                              s2.at[b]).wait()

    def issue_scatter(b):
        pltpu.async_copy(xlr2.at[b], acc_sh.at[sdst2.at[b]], s1.at[b],
                         add=True)
        pltpu.async_copy(av2.at[b], den_sh.at[sdst2.at[b]], s2.at[b],
                         add=True)

    def compute_chunk(b):
        for t in range(C // 16):
            sdst2[b, pl.ds(16 * t, 16)] = dstx2[b, pl.ds(16 * t, 16)]

        @functools.partial(plsc.parallel_loop, 0, NGROUP, unroll=2)
        def group_body(j):
            for k in range(G):
                row = j * G + k
                acc = None
                for q in range(8):
                    sq = (xlr2[b, row, pl.ds(16 * q, 16)] +
                          xrr2[b, row, pl.ds(16 * q, 16)])
                    eq = jnp.maximum(sq, 0.2 * sq)
                    t = eq * att_regs[q]
                    acc = t if acc is None else acc + t
                pbuf[b, j, k, :] = acc
            cols = [plsc.load_gather(pbuf.at[b, j], [iota16, splats[jc]])
                    for jc in range(16)]
            while len(cols) > 1:
                cols = [cols[i] + cols[i + 1] for i in range(0, len(cols), 2)]
            aexp = jnp.exp(cols[0])
            av2[b, pl.ds(j * G, 16)] = aexp
            for jc in range(16):
                plsc.store_scatter(abuf.at[b, j], [iota16, splats[jc]], aexp)
            for k in range(G):
                row = j * G + k
                a16 = abuf[b, j, k, :]
                for q in range(8):
                    xlr2[b, row, pl.ds(16 * q, 16)] = (
                        xlr2[b, row, pl.ds(16 * q, 16)] * a16)

    issue_idx(0, 0)
    wait_idx(0)
    issue_idx(1, 1)
    issue_gathers(0)

    def chunk_body(i, carry):
        b = lax.rem(i, 2)
        nb = 1 - b

        @pl.when(i + 1 < NCHUNK)
        def _():
            wait_idx(nb)

        @pl.when(i > 0)
        def _():
            wait_scatter(nb)

        @pl.when(i + 1 < NCHUNK)
        def _():
            issue_gathers(nb)

        wait_gathers(b)

        @pl.when(b == 0)
        def _():
            compute_chunk(0)

        @pl.when(b == 1)
        def _():
            compute_chunk(1)

        issue_scatter(b)

        @pl.when(i + 2 < NCHUNK)
        def _():
            issue_idx(i + 2, b)
        return carry

    lax.fori_loop(0, NCHUNK, chunk_body, 0)
    wait_scatter(lax.rem(NCHUNK - 1, 2))

    plsc.subcore_barrier()
    pltpu.sync_copy(acc_sh.at[pl.ds(s * ROWS_PT, ROWS_PT)],
                    out_hbm.at[c, pl.ds(s * ROWS_PT, ROWS_PT)])
    pltpu.sync_copy(den_sh.at[pl.ds(s * ROWS_PT, ROWS_PT)],
                    den_hbm.at[c, pl.ds(s * ROWS_PT, ROWS_PT)])

    @pl.when(s == NS - 1)
    def _():
        pltpu.sync_copy(acc_sh.at[pl.ds(NS * ROWS_PT, ROWS_TAIL)],
                        out_hbm.at[c, pl.ds(NS * ROWS_PT, ROWS_TAIL)])
        pltpu.sync_copy(den_sh.at[pl.ds(NS * ROWS_PT, ROWS_TAIL)],
                        den_hbm.at[c, pl.ds(NS * ROWS_PT, ROWS_TAIL)])


_edge_pass = functools.partial(
    pl.kernel,
    out_type=(jax.ShapeDtypeStruct((NC, N, D), jnp.float32),
              jax.ShapeDtypeStruct((NC, N), jnp.float32)),
    mesh=_MESH,
    scratch_types=[
        pltpu.VMEM_SHARED((N, D), jnp.float32),
        pltpu.VMEM_SHARED((N,), jnp.float32),
        pltpu.VMEM((2, C, D), jnp.float32),
        pltpu.VMEM((2, C, D), jnp.float32),
        pltpu.VMEM((2, C), jnp.int32),
        pltpu.VMEM((2, C), jnp.int32),
        pltpu.VMEM((2, C), jnp.float32),
        pltpu.VMEM((2, C), jnp.int32),
        pltpu.VMEM((D,), jnp.float32),
        pltpu.VMEM((2, NGROUP, G, 16), jnp.float32),
        pltpu.VMEM((2, NGROUP, 16, 16), jnp.float32),
        pltpu.SemaphoreType.DMA((2,)),
        pltpu.SemaphoreType.DMA((2,)),
        pltpu.SemaphoreType.DMA((2,)),
        pltpu.SemaphoreType.DMA((2,)),
        pltpu.SemaphoreType.DMA((2,)),
    ],
    compiler_params=pltpu.CompilerParams(needs_layout_passes=False,
                                         use_tc_tiling_on_sc=False),
)(_edge_body)


def _mm(a, b):
    return jax.lax.dot(a, b)


_BR = 1000
_NBLK = N // _BR


def _full(shape):
    return pl.BlockSpec(shape, lambda i: tuple(0 for _ in shape))


def _rows(width):
    return pl.BlockSpec((_BR, width), lambda i: (i, 0))


def _tc_pre_body(x_ref, Win_ref, bin_ref, Wl_ref, bl_ref, Wr_ref, br_ref,
                 xl_ref, xr_ref):
    h = jnp.maximum(_mm(x_ref[...], Win_ref[...]) + bin_ref[...], 0.0)
    xl_ref[...] = _mm(h, Wl_ref[...]) + bl_ref[...]
    xr_ref[...] = _mm(h, Wr_ref[...]) + br_ref[...]


def _tc_pre(x, W_in, b_in, Wl, bl, Wr, br):
    return pl.pallas_call(
        _tc_pre_body,
        grid=(_NBLK,),
        in_specs=[_rows(D), _full((D, D)), _full((1, D)), _full((D, D)),
                  _full((1, D)), _full((D, D)), _full((1, D))],
        out_specs=[_rows(D), _rows(D)],
        out_shape=[jax.ShapeDtypeStruct((N, D), jnp.float32)] * 2,
    )(x, W_in, b_in.reshape(1, D), Wl, bl.reshape(1, D), Wr, br.reshape(1, D))


def _normalize(acc_ref, den_ref, bias_ref, g_ref, bt_ref, res, do_relu):
    num = acc_ref[0] + acc_ref[1]
    den = den_ref[0] + den_ref[1]
    h = num / (den + 1e-16) + bias_ref[...]
    if res is not None:
        h = h + res
    mu = jnp.mean(h, axis=-1, keepdims=True)
    var = jnp.mean((h - mu) ** 2, axis=-1, keepdims=True)
    h = (h - mu) / jnp.sqrt(var + 1e-5) * g_ref[...] + bt_ref[...]
    if do_relu:
        h = jnp.maximum(h, 0.0)
    return h


def _tc_mid_body(has_res, args):
    if has_res:
        (acc_ref, den_ref, bias_ref, g_ref, bt_ref, res_ref, Wl_ref, bl_ref,
         Wr_ref, br_ref, h_ref, xl_ref, xr_ref) = args
        res = res_ref[...]
    else:
        (acc_ref, den_ref, bias_ref, g_ref, bt_ref, Wl_ref, bl_ref, Wr_ref,
         br_ref, h_ref, xl_ref, xr_ref) = args
        res = None
    h = _normalize(acc_ref, den_ref, bias_ref, g_ref, bt_ref, res, True)
    h_ref[...] = h
    xl_ref[...] = _mm(h, Wl_ref[...]) + bl_ref[...]
    xr_ref[...] = _mm(h, Wr_ref[...]) + br_ref[...]


_ACC_SPEC = pl.BlockSpec((NC, _BR, D), lambda i: (0, i, 0))
_DEN_SPEC = pl.BlockSpec((NC, _BR, 1), lambda i: (0, i, 0))


def _tc_mid(acc, den, bias, g, bt, res, Wl, bl, Wr, br):
    has_res = res is not None
    in_specs = [_ACC_SPEC, _DEN_SPEC,
                _full((1, D)), _full((1, D)), _full((1, D))]
    ops = [acc, den.reshape(NC, N, 1), bias.reshape(1, D), g.reshape(1, D),
           bt.reshape(1, D)]
    if has_res:
        in_specs.append(_rows(D))
        ops.append(res)
    in_specs += [_full((D, D)), _full((1, D)), _full((D, D)), _full((1, D))]
    ops += [Wl, bl.reshape(1, D), Wr, br.reshape(1, D)]
    return pl.pallas_call(
        lambda *args: _tc_mid_body(has_res, args),
        grid=(_NBLK,),
        in_specs=in_specs,
        out_specs=[_rows(D)] * 3,
        out_shape=[jax.ShapeDtypeStruct((N, D), jnp.float32)] * 3,
    )(*ops)


def _tc_post_body(acc_ref, den_ref, bias_ref, g_ref, bt_ref, res_ref,
                  Wo1_ref, bo1_ref, Wo2_ref, bo2_ref, out_ref):
    h = _normalize(acc_ref, den_ref, bias_ref, g_ref, bt_ref, res_ref[...],
                   False)
    t = jnp.maximum(_mm(h, Wo1_ref[...]) + bo1_ref[...], 0.0)
    out_ref[...] = _mm(t, Wo2_ref[...]) + bo2_ref[...]


def _tc_post(acc, den, bias, g, bt, res, Wo1, bo1, Wo2, bo2):
    return pl.pallas_call(
        _tc_post_body,
        grid=(_NBLK,),
        in_specs=[_ACC_SPEC, _DEN_SPEC,
                  _full((1, D)), _full((1, D)), _full((1, D)), _rows(D),
                  _full((D, D // 2)), _full((1, D // 2)),
                  _full((D // 2, 2)), _full((1, 2))],
        out_specs=_rows(2),
        out_shape=jax.ShapeDtypeStruct((N, 2), jnp.float32),
    )(acc, den.reshape(NC, N, 1), bias.reshape(1, D), g.reshape(1, D),
      bt.reshape(1, D), res, Wo1, bo1.reshape(1, D // 2), Wo2,
      bo2.reshape(1, 2))


def kernel(x, edge_index, W_in, b_in, Wl0, bl0, Wr0, br0, att0, bias0, g0, bt0,
           Wl1, bl1, Wr1, br1, att1, bias1, g1, bt1,
           Wl2, bl2, Wr2, br2, att2, bias2, g2, bt2, Wo1, bo1, Wo2, bo2):
    src = edge_index[0]
    dst = edge_index[1]
    zeros = jnp.zeros((N, D), jnp.float32)
    zn = jnp.zeros((N,), jnp.float32)

    xl, xr = _tc_pre(x, W_in, b_in, Wl0, bl0, Wr0, br0)
    acc0, den0 = _edge_pass(xl, xr, src, dst, att0[0], zeros, zn)
    h1, xl1, xr1 = _tc_mid(acc0, den0, bias0, g0, bt0, None, Wl1, bl1, Wr1, br1)
    acc1, den1 = _edge_pass(xl1, xr1, src, dst, att1[0], zeros, zn)
    h2, xl2, xr2 = _tc_mid(acc1, den1, bias1, g1, bt1, h1, Wl2, bl2, Wr2, br2)
    acc2, den2 = _edge_pass(xl2, xr2, src, dst, att2[0], zeros, zn)
    return _tc_post(acc2, den2, bias2, g2, bt2, h2, Wo1, bo1, Wo2, bo2)
